# baseline (speedup 1.0000x reference)
import functools
import math

import jax
import jax.numpy as jnp
from jax import lax
from jax.experimental import pallas as pl
from jax.experimental.pallas import tpu as pltpu
from jax.experimental.pallas import tpu_sc as plsc

_S, _B, _H = 200, 4096, 64
_K = 4
_NW = 32
_BPW = _B // _NW
_LANES = 16
_BBLK_A = 128
_BBLK_C = 128


def _scores_body(q_ref, wq_ref, wk_ref, mem_ref, s_ref, st_ref):
    q = jnp.dot(q_ref[...], wq_ref[...].T,
                preferred_element_type=jnp.float32)
    mem = mem_ref[...]
    keys = jnp.dot(mem.reshape(_S * _BBLK_A, _H), wk_ref[...].T,
                   preferred_element_type=jnp.float32).reshape(_S, _BBLK_A, _H)
    scale = 1.0 / math.sqrt(_H)
    st = jnp.sum(q[None, :, :] * keys, axis=-1) * scale
    st_ref[...] = st
    s_ref[...] = st.T


def _fuse_body(q_ref, attn_ref, idx_ref, mem_ref,
               wvT_ref, w1qT_ref, w1rT_ref, b1_ref, w2T_ref, b2_ref,
               woT_ref, bo_ref, out_ref, gate_ref):
    q = q_ref[...]
    attn = attn_ref[...]
    idx = idx_ref[...]
    mem = mem_ref[...]
    iota_s = lax.broadcasted_iota(jnp.int32, (q.shape[0], _S), 1)
    w = jnp.zeros((q.shape[0], _S), dtype=jnp.float32)
    for k in range(_K):
        w = w + jnp.where(iota_s == idx[:, k:k + 1], attn[:, k:k + 1], 0.0)
    r_raw = jnp.sum(w.T[:, :, None] * mem, axis=0)
    r = jnp.dot(r_raw, wvT_ref[...], preferred_element_type=jnp.float32)
    h1 = jnp.dot(q, w1qT_ref[...], preferred_element_type=jnp.float32)
    h1 = h1 + jnp.dot(r, w1rT_ref[...], preferred_element_type=jnp.float32)
    h1 = jnp.maximum(h1 + b1_ref[...], 0.0)
    gate = jax.nn.sigmoid(
        jnp.dot(h1, w2T_ref[...], preferred_element_type=jnp.float32)
        + b2_ref[...])
    fused = gate * q + (1.0 - gate) * r
    out_ref[...] = (jnp.dot(fused, woT_ref[...],
                            preferred_element_type=jnp.float32) + bo_ref[...])
    gate_ref[...] = gate


def _sc_retrieve_body(stT_hbm, attn_out, idx_out, st_v, attn_v, idxv_v):
    cid = lax.axis_index("c")
    sid = lax.axis_index("s")
    wid = sid * 2 + cid
    base = wid * _BPW
    pltpu.sync_copy(stT_hbm.at[:, pl.ds(base, _BPW)], st_v)
    lane = lax.iota(jnp.int32, _LANES)
    neg = jnp.full((_LANES,), -jnp.inf, dtype=jnp.float32)
    zero = jnp.zeros((_LANES,), dtype=jnp.int32)
    for j in range(_BPW // _LANES):
        col0 = j * _LANES

        def body(s, carry, col0=col0):
            t0, t1, t2, t3, i0, i1, i2, i3 = carry
            v = st_v[s, pl.ds(col0, _LANES)]
            sv = jnp.full((_LANES,), s, dtype=jnp.int32)
            c0 = v > t0
            c1 = v > t1
            c2 = v > t2
            c3 = v > t3
            n0 = jnp.where(c0, v, t0)
            n1 = jnp.where(c0, t0, jnp.where(c1, v, t1))
            n2 = jnp.where(c1, t1, jnp.where(c2, v, t2))
            n3 = jnp.where(c2, t2, jnp.where(c3, v, t3))
            m0 = jnp.where(c0, sv, i0)
            m1 = jnp.where(c0, i0, jnp.where(c1, sv, i1))
            m2 = jnp.where(c1, i1, jnp.where(c2, sv, i2))
            m3 = jnp.where(c2, i2, jnp.where(c3, sv, i3))
            return (n0, n1, n2, n3, m0, m1, m2, m3)

        t0, t1, t2, t3, i0, i1, i2, i3 = lax.fori_loop(
            0, _S, body, (neg, neg, neg, neg, zero, zero, zero, zero))
        e1 = jnp.exp(t1 - t0)
        e2 = jnp.exp(t2 - t0)
        e3 = jnp.exp(t3 - t0)
        inv = 1.0 / (1.0 + e1 + e2 + e3)
        a0 = inv
        a1 = e1 * inv
        a2 = e2 * inv
        a3 = e3 * inv
        for k, (ak, ik) in enumerate(((a0, i0), (a1, i1), (a2, i2), (a3, i3))):
            attn_v[k, pl.ds(col0, _LANES)] = ak
            idxv_v[k, pl.ds(col0, _LANES)] = ik
    pltpu.sync_copy(attn_v, attn_out.at[:, pl.ds(base, _BPW)])
    pltpu.sync_copy(idxv_v, idx_out.at[:, pl.ds(base, _BPW)])


def kernel(query, cached_memory, Wq, Wk, Wv, Wg1, bg1, Wg2, bg2, Wo, bo):
    S, B, H = cached_memory.shape
    f32 = jnp.float32

    scores, scoresT = pl.pallas_call(
        _scores_body,
        grid=(B // _BBLK_A,),
        in_specs=[
            pl.BlockSpec((_BBLK_A, H), lambda i: (i, 0)),
            pl.BlockSpec((H, H), lambda i: (0, 0)),
            pl.BlockSpec((H, H), lambda i: (0, 0)),
            pl.BlockSpec((S, _BBLK_A, H), lambda i: (0, i, 0)),
        ],
        out_specs=[
            pl.BlockSpec((_BBLK_A, S), lambda i: (i, 0)),
            pl.BlockSpec((S, _BBLK_A), lambda i: (0, i)),
        ],
        out_shape=[
            jax.ShapeDtypeStruct((B, S), f32),
            jax.ShapeDtypeStruct((S, B), f32),
        ],
    )(query, Wq, Wk, cached_memory)

    mesh = plsc.VectorSubcoreMesh(core_axis_name="c", subcore_axis_name="s")
    sc_retrieve = functools.partial(
        pl.kernel,
        out_type=[
            jax.ShapeDtypeStruct((_K, B), f32),
            jax.ShapeDtypeStruct((_K, B), jnp.int32),
        ],
        mesh=mesh,
        scratch_types=[
            pltpu.VMEM((S, _BPW), f32),
            pltpu.VMEM((_K, _BPW), f32),
            pltpu.VMEM((_K, _BPW), jnp.int32),
        ],
    )(_sc_retrieve_body)
    attn_kB, idx_kB = sc_retrieve(scoresT)
    attn2 = attn_kB.T
    idx2 = idx_kB.T

    w1q = Wg1[:, :H] + Wg1[:, 2 * H:]
    w1r = Wg1[:, H:2 * H] - Wg1[:, 2 * H:]
    blk = pl.BlockSpec((_BBLK_C, H), lambda i: (i, 0))
    wblk = pl.BlockSpec((H, H), lambda i: (0, 0))
    bblk = pl.BlockSpec((1, H), lambda i: (0, 0))
    out, gate = pl.pallas_call(
        _fuse_body,
        grid=(B // _BBLK_C,),
        in_specs=[
            blk,
            pl.BlockSpec((_BBLK_C, _K), lambda i: (i, 0)),
            pl.BlockSpec((_BBLK_C, _K), lambda i: (i, 0)),
            pl.BlockSpec((S, _BBLK_C, H), lambda i: (0, i, 0)),
            wblk, wblk, wblk, bblk, wblk, bblk, wblk, bblk,
        ],
        out_specs=[blk, blk],
        out_shape=[
            jax.ShapeDtypeStruct((B, H), f32),
            jax.ShapeDtypeStruct((B, H), f32),
        ],
    )(query, attn2, idx2, cached_memory,
      Wv.T, w1q.T, w1r.T, bg1.reshape(1, H), Wg2.T, bg2.reshape(1, H),
      Wo.T, bo.reshape(1, H))

    return (out, attn2, scores, idx2, gate)


# speedup vs baseline: 3.8132x; 3.8132x over previous
import functools
import math

import jax
import jax.numpy as jnp
from jax import lax
from jax.experimental import pallas as pl
from jax.experimental.pallas import tpu as pltpu
from jax.experimental.pallas import tpu_sc as plsc

_S, _B, _H = 200, 4096, 64
_K = 4
_NW = 32
_BPW = _B // _NW
_LANES = 16
_BBLK_A = 128
_BBLK_C = 128


def _scores_body(qT_ref, wq_ref, wk_ref, memT_ref, st_ref):
    qT = jnp.dot(wq_ref[...], qT_ref[...],
                 preferred_element_type=jnp.float32)
    memT = memT_ref[...]
    wk_b = jnp.broadcast_to(wk_ref[...][None], (_S, _H, _H))
    keysT = jax.lax.dot_general(
        wk_b, memT, (((2,), (1,)), ((0,), (0,))),
        preferred_element_type=jnp.float32)
    scale = 1.0 / math.sqrt(_H)
    st_ref[...] = jnp.sum(keysT * qT[None], axis=1) * scale


def _fuse_body(qT_ref, attnK_ref, idxK_ref, memT_ref,
               wv_ref, w1q_ref, w1r_ref, b1_ref, w2_ref, b2_ref,
               wo_ref, bo_ref, outT_ref, gateT_ref):
    qT = qT_ref[...]
    attnK = attnK_ref[...]
    idxK = idxK_ref[...]
    memT = memT_ref[...]
    iota_s = lax.broadcasted_iota(jnp.int32, (_S, qT.shape[1]), 0)
    wT = jnp.zeros((_S, qT.shape[1]), dtype=jnp.float32)
    for k in range(_K):
        wT = wT + jnp.where(iota_s == idxK[k:k + 1, :], attnK[k:k + 1, :], 0.0)
    r_rawT = jnp.sum(wT[:, None, :] * memT, axis=0)
    rT = jnp.dot(wv_ref[...], r_rawT,
                 preferred_element_type=jnp.float32)
    h1 = jnp.dot(w1q_ref[...], qT, preferred_element_type=jnp.float32)
    h1 = h1 + jnp.dot(w1r_ref[...], rT, preferred_element_type=jnp.float32)
    h1 = jnp.maximum(h1 + b1_ref[...], 0.0)
    gateT = jax.nn.sigmoid(
        jnp.dot(w2_ref[...], h1, preferred_element_type=jnp.float32)
        + b2_ref[...])
    fusedT = gateT * qT + (1.0 - gateT) * rT
    outT_ref[...] = (jnp.dot(wo_ref[...], fusedT,
                             preferred_element_type=jnp.float32) + bo_ref[...])
    gateT_ref[...] = gateT


def _sc_retrieve_body(stT_hbm, attn_out, idx_out, st_v, attn_v, idxv_v):
    cid = lax.axis_index("c")
    sid = lax.axis_index("s")
    wid = sid * 2 + cid
    base = wid * _BPW
    pltpu.sync_copy(stT_hbm.at[:, pl.ds(base, _BPW)], st_v)
    neg = jnp.full((_LANES,), -jnp.inf, dtype=jnp.float32)
    zero = jnp.zeros((_LANES,), dtype=jnp.int32)
    for j in range(_BPW // _LANES):
        col0 = j * _LANES

        def body(s, carry, col0=col0):
            t0, t1, t2, t3, i0, i1, i2, i3 = carry
            v = st_v[s, pl.ds(col0, _LANES)]
            sv = jnp.full((_LANES,), s, dtype=jnp.int32)
            c0 = v > t0
            c1 = v > t1
            c2 = v > t2
            c3 = v > t3
            n0 = jnp.where(c0, v, t0)
            n1 = jnp.where(c0, t0, jnp.where(c1, v, t1))
            n2 = jnp.where(c1, t1, jnp.where(c2, v, t2))
            n3 = jnp.where(c2, t2, jnp.where(c3, v, t3))
            m0 = jnp.where(c0, sv, i0)
            m1 = jnp.where(c0, i0, jnp.where(c1, sv, i1))
            m2 = jnp.where(c1, i1, jnp.where(c2, sv, i2))
            m3 = jnp.where(c2, i2, jnp.where(c3, sv, i3))
            return (n0, n1, n2, n3, m0, m1, m2, m3)

        t0, t1, t2, t3, i0, i1, i2, i3 = lax.fori_loop(
            0, _S, body, (neg, neg, neg, neg, zero, zero, zero, zero))
        e1 = jnp.exp(t1 - t0)
        e2 = jnp.exp(t2 - t0)
        e3 = jnp.exp(t3 - t0)
        inv = 1.0 / (1.0 + e1 + e2 + e3)
        a0 = inv
        a1 = e1 * inv
        a2 = e2 * inv
        a3 = e3 * inv
        for k, (ak, ik) in enumerate(((a0, i0), (a1, i1), (a2, i2), (a3, i3))):
            attn_v[k, pl.ds(col0, _LANES)] = ak
            idxv_v[k, pl.ds(col0, _LANES)] = ik
    pltpu.sync_copy(attn_v, attn_out.at[:, pl.ds(base, _BPW)])
    pltpu.sync_copy(idxv_v, idx_out.at[:, pl.ds(base, _BPW)])


def kernel(query, cached_memory, Wq, Wk, Wv, Wg1, bg1, Wg2, bg2, Wo, bo):
    S, B, H = cached_memory.shape
    f32 = jnp.float32
    queryT = query.T
    memT = jnp.transpose(cached_memory, (0, 2, 1))

    qblk = pl.BlockSpec((H, _BBLK_A), lambda i: (0, i))
    wblk = pl.BlockSpec((H, H), lambda i: (0, 0))
    scoresT = pl.pallas_call(
        _scores_body,
        grid=(B // _BBLK_A,),
        in_specs=[
            qblk, wblk, wblk,
            pl.BlockSpec((S, H, _BBLK_A), lambda i: (0, 0, i)),
        ],
        out_specs=pl.BlockSpec((S, _BBLK_A), lambda i: (0, i)),
        out_shape=jax.ShapeDtypeStruct((S, B), f32),
    )(queryT, Wq, Wk, memT)

    mesh = plsc.VectorSubcoreMesh(core_axis_name="c", subcore_axis_name="s")
    sc_retrieve = functools.partial(
        pl.kernel,
        out_type=[
            jax.ShapeDtypeStruct((_K, B), f32),
            jax.ShapeDtypeStruct((_K, B), jnp.int32),
        ],
        mesh=mesh,
        scratch_types=[
            pltpu.VMEM((S, _BPW), f32),
            pltpu.VMEM((_K, _BPW), f32),
            pltpu.VMEM((_K, _BPW), jnp.int32),
        ],
    )(_sc_retrieve_body)
    attn_kB, idx_kB = sc_retrieve(scoresT)

    w1q = Wg1[:, :H] + Wg1[:, 2 * H:]
    w1r = Wg1[:, H:2 * H] - Wg1[:, 2 * H:]
    kblk = pl.BlockSpec((_K, _BBLK_C), lambda i: (0, i))
    qblk_c = pl.BlockSpec((H, _BBLK_C), lambda i: (0, i))
    bblk = pl.BlockSpec((H, 1), lambda i: (0, 0))
    outT, gateT = pl.pallas_call(
        _fuse_body,
        grid=(B // _BBLK_C,),
        in_specs=[
            qblk_c, kblk, kblk,
            pl.BlockSpec((S, H, _BBLK_C), lambda i: (0, 0, i)),
            wblk, wblk, wblk, bblk, wblk, bblk, wblk, bblk,
        ],
        out_specs=[qblk_c, qblk_c],
        out_shape=[
            jax.ShapeDtypeStruct((H, B), f32),
            jax.ShapeDtypeStruct((H, B), f32),
        ],
    )(queryT, attn_kB, idx_kB, memT,
      Wv, w1q, w1r, bg1.reshape(H, 1), Wg2, bg2.reshape(H, 1),
      Wo, bo.reshape(H, 1))

    return (outT.T, attn_kB.T, scoresT.T, idx_kB.T, gateT.T)


# speedup vs baseline: 3.9823x; 1.0443x over previous
import functools
import math

import jax
import jax.numpy as jnp
from jax import lax
from jax.experimental import pallas as pl
from jax.experimental.pallas import tpu as pltpu
from jax.experimental.pallas import tpu_sc as plsc

_S, _B, _H = 200, 4096, 64
_K = 4
_NW = 32
_BPW = _B // _NW
_LANES = 16
_BBLK_A = 256
_BBLK_C = 256


def _scores_body(qT_ref, wq_ref, wk_ref, memT_ref, st_ref):
    qT = jnp.dot(wq_ref[...], qT_ref[...],
                 preferred_element_type=jnp.float32)
    memT = memT_ref[...]
    wk_b = jnp.broadcast_to(wk_ref[...][None], (_S, _H, _H))
    keysT = jax.lax.dot_general(
        wk_b, memT, (((2,), (1,)), ((0,), (0,))),
        preferred_element_type=jnp.float32)
    scale = 1.0 / math.sqrt(_H)
    st_ref[...] = jnp.sum(keysT * qT[None], axis=1) * scale


def _fuse_body(qT_ref, attnK_ref, idxK_ref, memT_ref,
               wv_ref, w1q_ref, w1r_ref, b1_ref, w2_ref, b2_ref,
               wo_ref, bo_ref, outT_ref, gateT_ref):
    qT = qT_ref[...]
    attnK = attnK_ref[...]
    idxK = idxK_ref[...]
    memT = memT_ref[...]
    iota_s = lax.broadcasted_iota(jnp.int32, (_S, qT.shape[1]), 0)
    wT = jnp.zeros((_S, qT.shape[1]), dtype=jnp.float32)
    for k in range(_K):
        wT = wT + jnp.where(iota_s == idxK[k:k + 1, :], attnK[k:k + 1, :], 0.0)
    r_rawT = jnp.sum(wT[:, None, :] * memT, axis=0)
    rT = jnp.dot(wv_ref[...], r_rawT,
                 preferred_element_type=jnp.float32)
    h1 = jnp.dot(w1q_ref[...], qT, preferred_element_type=jnp.float32)
    h1 = h1 + jnp.dot(w1r_ref[...], rT, preferred_element_type=jnp.float32)
    h1 = jnp.maximum(h1 + b1_ref[...], 0.0)
    gateT = jax.nn.sigmoid(
        jnp.dot(w2_ref[...], h1, preferred_element_type=jnp.float32)
        + b2_ref[...])
    fusedT = gateT * qT + (1.0 - gateT) * rT
    outT_ref[...] = (jnp.dot(wo_ref[...], fusedT,
                             preferred_element_type=jnp.float32) + bo_ref[...])
    gateT_ref[...] = gateT


def _sc_retrieve_body(stT_hbm, attn_out, idx_out, st_v, attn_v, idxv_v):
    cid = lax.axis_index("c")
    sid = lax.axis_index("s")
    wid = sid * 2 + cid
    base = wid * _BPW
    pltpu.sync_copy(stT_hbm.at[:, pl.ds(base, _BPW)], st_v)
    neg = jnp.full((_LANES,), -jnp.inf, dtype=jnp.float32)
    zero = jnp.zeros((_LANES,), dtype=jnp.int32)
    for j in range(_BPW // _LANES):
        col0 = j * _LANES

        def body(s, carry, col0=col0):
            t0, t1, t2, t3, i0, i1, i2, i3 = carry
            v = st_v[s, pl.ds(col0, _LANES)]
            sv = jnp.full((_LANES,), s, dtype=jnp.int32)
            c0 = v > t0
            c1 = v > t1
            c2 = v > t2
            c3 = v > t3
            n0 = jnp.where(c0, v, t0)
            n1 = jnp.where(c0, t0, jnp.where(c1, v, t1))
            n2 = jnp.where(c1, t1, jnp.where(c2, v, t2))
            n3 = jnp.where(c2, t2, jnp.where(c3, v, t3))
            m0 = jnp.where(c0, sv, i0)
            m1 = jnp.where(c0, i0, jnp.where(c1, sv, i1))
            m2 = jnp.where(c1, i1, jnp.where(c2, sv, i2))
            m3 = jnp.where(c2, i2, jnp.where(c3, sv, i3))
            return (n0, n1, n2, n3, m0, m1, m2, m3)

        t0, t1, t2, t3, i0, i1, i2, i3 = lax.fori_loop(
            0, _S, body, (neg, neg, neg, neg, zero, zero, zero, zero),
            unroll=4)
        e1 = jnp.exp(t1 - t0)
        e2 = jnp.exp(t2 - t0)
        e3 = jnp.exp(t3 - t0)
        inv = 1.0 / (1.0 + e1 + e2 + e3)
        a0 = inv
        a1 = e1 * inv
        a2 = e2 * inv
        a3 = e3 * inv
        for k, (ak, ik) in enumerate(((a0, i0), (a1, i1), (a2, i2), (a3, i3))):
            attn_v[k, pl.ds(col0, _LANES)] = ak
            idxv_v[k, pl.ds(col0, _LANES)] = ik
    pltpu.sync_copy(attn_v, attn_out.at[:, pl.ds(base, _BPW)])
    pltpu.sync_copy(idxv_v, idx_out.at[:, pl.ds(base, _BPW)])


def kernel(query, cached_memory, Wq, Wk, Wv, Wg1, bg1, Wg2, bg2, Wo, bo):
    S, B, H = cached_memory.shape
    f32 = jnp.float32
    queryT = query.T
    memT = jnp.transpose(cached_memory, (0, 2, 1))

    qblk = pl.BlockSpec((H, _BBLK_A), lambda i: (0, i))
    wblk = pl.BlockSpec((H, H), lambda i: (0, 0))
    scoresT = pl.pallas_call(
        _scores_body,
        grid=(B // _BBLK_A,),
        in_specs=[
            qblk, wblk, wblk,
            pl.BlockSpec((S, H, _BBLK_A), lambda i: (0, 0, i)),
        ],
        out_specs=pl.BlockSpec((S, _BBLK_A), lambda i: (0, i)),
        out_shape=jax.ShapeDtypeStruct((S, B), f32),
    )(queryT, Wq, Wk, memT)

    mesh = plsc.VectorSubcoreMesh(core_axis_name="c", subcore_axis_name="s")
    sc_retrieve = functools.partial(
        pl.kernel,
        out_type=[
            jax.ShapeDtypeStruct((_K, B), f32),
            jax.ShapeDtypeStruct((_K, B), jnp.int32),
        ],
        mesh=mesh,
        scratch_types=[
            pltpu.VMEM((S, _BPW), f32),
            pltpu.VMEM((_K, _BPW), f32),
            pltpu.VMEM((_K, _BPW), jnp.int32),
        ],
    )(_sc_retrieve_body)
    attn_kB, idx_kB = sc_retrieve(scoresT)

    w1q = Wg1[:, :H] + Wg1[:, 2 * H:]
    w1r = Wg1[:, H:2 * H] - Wg1[:, 2 * H:]
    kblk = pl.BlockSpec((_K, _BBLK_C), lambda i: (0, i))
    qblk_c = pl.BlockSpec((H, _BBLK_C), lambda i: (0, i))
    bblk = pl.BlockSpec((H, 1), lambda i: (0, 0))
    outT, gateT = pl.pallas_call(
        _fuse_body,
        grid=(B // _BBLK_C,),
        in_specs=[
            qblk_c, kblk, kblk,
            pl.BlockSpec((S, H, _BBLK_C), lambda i: (0, 0, i)),
            wblk, wblk, wblk, bblk, wblk, bblk, wblk, bblk,
        ],
        out_specs=[qblk_c, qblk_c],
        out_shape=[
            jax.ShapeDtypeStruct((H, B), f32),
            jax.ShapeDtypeStruct((H, B), f32),
        ],
    )(queryT, attn_kB, idx_kB, memT,
      Wv, w1q, w1r, bg1.reshape(H, 1), Wg2, bg2.reshape(H, 1),
      Wo, bo.reshape(H, 1))

    return (outT.T, attn_kB.T, scoresT.T, idx_kB.T, gateT.T)
